# baseline (device time: 12812 ns/iter reference)
import jax
import jax.numpy as jnp
from jax import lax
from jax.experimental import pallas as pl
from jax.experimental.pallas import tpu as pltpu

N_DEV = 4
C = 4


def kernel(x):
    m, n = x.shape
    h = m // 2
    rows = h // C
    nchunks = 2 * C

    def body(x_ref, out_ref, sbuf, tbuf, r1buf, r2buf,
             s1sems, r1sems, s2sems, r2sems):
        my = lax.axis_index("i")
        p1 = my ^ 1
        p2 = 3 - my

        barrier_sem = pltpu.get_barrier_semaphore()
        for nbr in (p1, p2):
            pl.semaphore_signal(
                barrier_sem, inc=1,
                device_id=(nbr,), device_id_type=pl.DeviceIdType.MESH,
            )
        issue_order = [a * C + c for c in range(C) for a in (0, 1)]
        r1 = [None] * nchunks
        first = issue_order[:2]
        for k in first:
            sbuf[k] = x_ref[pl.ds(k * rows, rows), :].astype(jnp.bfloat16)
        pl.semaphore_wait(barrier_sem, 2)
        for k in issue_order:
            if k not in first:
                sbuf[k] = x_ref[pl.ds(k * rows, rows), :].astype(jnp.bfloat16)
            dev = p1 if k < C else p2
            rd = pltpu.make_async_remote_copy(
                src_ref=sbuf.at[k], dst_ref=r1buf.at[k],
                send_sem=s1sems.at[k], recv_sem=r1sems.at[k],
                device_id=(dev,), device_id_type=pl.DeviceIdType.MESH,
            )
            rd.start()
            r1[k] = rd

        order = issue_order
        r2 = [None] * nchunks
        for k in order:
            r1[k].wait_recv()
            tbuf[k] = sbuf[k] + r1buf[k]
            dev = p2 if k < C else p1
            rd = pltpu.make_async_remote_copy(
                src_ref=tbuf.at[k], dst_ref=r2buf.at[k],
                send_sem=s2sems.at[k], recv_sem=r2sems.at[k],
                device_id=(dev,), device_id_type=pl.DeviceIdType.MESH,
            )
            rd.start()
            r2[k] = rd
        for k in order:
            r2[k].wait_recv()
            out_ref[pl.ds(k * rows, rows), :] = tbuf[k] + r2buf[k]

        for rd in r1:
            rd.wait_send()
        for rd in r2:
            rd.wait_send()

    chunk_vmem = pltpu.VMEM((nchunks, rows, n), jnp.bfloat16)
    return pl.pallas_call(
        body,
        out_shape=jax.ShapeDtypeStruct((m, n), jnp.bfloat16),
        in_specs=[pl.BlockSpec(memory_space=pltpu.VMEM)],
        out_specs=pl.BlockSpec(memory_space=pltpu.VMEM),
        scratch_shapes=[
            chunk_vmem,
            chunk_vmem,
            chunk_vmem,
            chunk_vmem,
            pltpu.SemaphoreType.DMA((nchunks,)),
            pltpu.SemaphoreType.DMA((nchunks,)),
            pltpu.SemaphoreType.DMA((nchunks,)),
            pltpu.SemaphoreType.DMA((nchunks,)),
        ],
        compiler_params=pltpu.CompilerParams(collective_id=0),
    )(x)
